# baseline (speedup 1.0000x reference)
import functools

import jax
import jax.numpy as jnp
from jax import lax
from jax.experimental import pallas as pl
from jax.experimental.pallas import tpu as pltpu
from jax.experimental.pallas import tpu_sc as plsc

N = 10000
E = 320000
EH = E // 2
F = 64
SW = 80
NC = 2
NS = 16
NW = NC * NS
PPW = EH // NW
EPW = E // NW
GCH = 1000
SCH = 1000
NPT = N // NS
N2 = 10240
NPT1 = N2 // NS

BN = 1000
BP = 6400


def _p0_body(x_ref, w_ref, a_ref, xn_ref):
    r = jnp.dot(x_ref[...], w_ref[...], preferred_element_type=jnp.float32)
    a_ref[...] = r[:, :F]
    xn_ref[...] = r[:, F:]


def _p0(x, w0):
    return pl.pallas_call(
        _p0_body,
        grid=(N // BN,),
        in_specs=[
            pl.BlockSpec((BN, 128), lambda i: (i, 0)),
            pl.BlockSpec((128, 128), lambda i: (0, 0)),
        ],
        out_specs=[
            pl.BlockSpec((BN, F), lambda i: (i, 0)),
            pl.BlockSpec((BN, F), lambda i: (i, 0)),
        ],
        out_shape=[
            jax.ShapeDtypeStruct((N, F), jnp.float32),
            jax.ShapeDtypeStruct((N, F), jnp.float32),
        ],
    )(x, w0)


def _p2_body(ag_ref, et_ref, w2_ref, ws_ref, hm_ref, c_ref, out_ref):
    et = et_ref[...]
    bp = jax.lax.dot_general(
        et, w2_ref[...], (((0,), (0,)), ((), ())),
        preferred_element_type=jnp.float32)
    wfull = jax.lax.dot_general(
        et, ws_ref[...], (((0,), (0,)), ((), ())),
        preferred_element_type=jnp.float32)
    h = ag_ref[...] + bp + c_ref[0:1, :]
    hm = hm_ref[...]
    mu = jnp.dot(h, hm, preferred_element_type=jnp.float32)
    hc = h - mu
    var = jnp.dot(hc * hc, hm, preferred_element_type=jnp.float32)
    hn = hc * lax.rsqrt(var + 1e-5) * c_ref[1:2, :] + c_ref[2:3, :]
    out_ref[...] = jnp.maximum(hn, 0.0) * wfull


def _p2(ag2, eawt, w2, wsel, hm, c8):
    return pl.pallas_call(
        _p2_body,
        grid=(EH // BP,),
        in_specs=[
            pl.BlockSpec((BP, 128), lambda i: (i, 0)),
            pl.BlockSpec((40, BP), lambda i: (0, i)),
            pl.BlockSpec((40, 128), lambda i: (0, 0)),
            pl.BlockSpec((40, 128), lambda i: (0, 0)),
            pl.BlockSpec((128, 128), lambda i: (0, 0)),
            pl.BlockSpec((8, 128), lambda i: (0, 0)),
        ],
        out_specs=pl.BlockSpec((BP, 128), lambda i: (i, 0)),
        out_shape=jax.ShapeDtypeStruct((EH, 128), jnp.float32),
    )(ag2, eawt, w2, wsel, hm, c8)


def _p4_body(s64_ref, sw_ref, sc_ref, xn_ref, oh_ref, up_ref, m_ref, v_ref,
             w2_ref, b2_ref, out_ref):
    sm = s64_ref[0] + s64_ref[1]
    wsum = sw_ref[0] + sw_ref[1]
    cnt = sc_ref[0] + sc_ref[1]
    recv = (
        jnp.dot(sm, m_ref[...], preferred_element_type=jnp.float32)
        + wsum * v_ref[0:1, :]
    ) / jnp.maximum(cnt, 1.0)
    h = (
        xn_ref[...] + recv
        + jnp.dot(oh_ref[...], up_ref[...], preferred_element_type=jnp.float32)
        + v_ref[1:2, :]
    )
    mu = jnp.mean(h, axis=-1, keepdims=True)
    var = jnp.mean((h - mu) ** 2, axis=-1, keepdims=True)
    hn = (h - mu) * lax.rsqrt(var + 1e-5) * v_ref[2:3, :] + v_ref[3:4, :]
    hr = jnp.maximum(hn, 0.0)
    out_ref[...] = (
        jnp.dot(hr, w2_ref[...], preferred_element_type=jnp.float32)
        + b2_ref[0:1, :]
    )


def _p4(s64, sw, sc, xn, oh, up, m64, v8, nw2, b2r, node_out):
    return pl.pallas_call(
        _p4_body,
        grid=(N // BN,),
        in_specs=[
            pl.BlockSpec((NC, BN, F), lambda i: (0, i, 0)),
            pl.BlockSpec((NC, BN, 1), lambda i: (0, i, 0)),
            pl.BlockSpec((NC, BN, 1), lambda i: (0, i, 0)),
            pl.BlockSpec((BN, F), lambda i: (i, 0)),
            pl.BlockSpec((BN, 16), lambda i: (i, 0)),
            pl.BlockSpec((16, F), lambda i: (0, 0)),
            pl.BlockSpec((F, F), lambda i: (0, 0)),
            pl.BlockSpec((8, F), lambda i: (0, 0)),
            pl.BlockSpec((F, node_out), lambda i: (0, 0)),
            pl.BlockSpec((8, node_out), lambda i: (0, 0)),
        ],
        out_specs=pl.BlockSpec((BN, node_out), lambda i: (i, 0)),
        out_shape=jax.ShapeDtypeStruct((N, node_out), jnp.float32),
    )(s64, sw, sc, xn, oh, up, m64, v8, nw2, b2r)


@functools.lru_cache(maxsize=None)
def _make_sc_gather():
    mesh = plsc.VectorSubcoreMesh(core_axis_name="c", subcore_axis_name="s")

    @functools.partial(
        pl.kernel,
        mesh=mesh,
        out_type=jax.ShapeDtypeStruct((EH, 128), jnp.float32),
        scratch_types=[
            pltpu.VMEM((GCH,), jnp.int32),
            pltpu.VMEM((GCH, F), jnp.float32),
            pltpu.SemaphoreType.DMA,
        ],
        compiler_params=pltpu.CompilerParams(use_tc_tiling_on_sc=False),
    )
    def _sc_gather(a_hbm, row_hbm, out_hbm, idx_v, rows_v, sem):
        cid = lax.axis_index("c")
        sid = lax.axis_index("s")
        wid = cid * NS + sid
        for ch in range(PPW // GCH):
            pbase = wid * PPW + ch * GCH
            for half in range(2):
                pltpu.sync_copy(row_hbm.at[pl.ds(half * EH + pbase, GCH)],
                                idx_v)
                pltpu.async_copy(a_hbm.at[idx_v], rows_v, sem).wait()
                pltpu.sync_copy(
                    rows_v,
                    out_hbm.at[pl.ds(pbase, GCH), pl.ds(half * F, F)])

    return _sc_gather


@functools.lru_cache(maxsize=None)
def _make_sc_scatter():
    mesh = plsc.VectorSubcoreMesh(core_axis_name="c", subcore_axis_name="s")

    @functools.partial(
        pl.kernel,
        mesh=mesh,
        out_type=[
            jax.ShapeDtypeStruct((NC, N, F), jnp.float32),
            jax.ShapeDtypeStruct((NC, N2), jnp.float32),
            jax.ShapeDtypeStruct((NC, N2), jnp.float32),
        ],
        scratch_types=[
            pltpu.VMEM((SCH,), jnp.int32),
            pltpu.VMEM((SCH, F), jnp.float32),
            pltpu.VMEM((SCH,), jnp.float32),
            pltpu.VMEM((SCH,), jnp.float32),
            pltpu.VMEM_SHARED((N, F), jnp.float32),
            pltpu.VMEM_SHARED((N2,), jnp.float32),
            pltpu.VMEM_SHARED((N2,), jnp.float32),
            pltpu.SemaphoreType.DMA,
        ],
        compiler_params=pltpu.CompilerParams(use_tc_tiling_on_sc=False),
    )
    def _sc_scatter(r_hbm, col_hbm, w_hbm, o64_hbm, ow_hbm, oc_hbm,
                    idx_v, rbuf_v, wvec_v, ones_v, a64_sh, aw_sh, ac_sh, sem):
        cid = lax.axis_index("c")
        sid = lax.axis_index("s")
        wid = cid * NS + sid

        def _zrow(i, _):
            for j in range(F // 16):
                rbuf_v[i, pl.ds(j * 16, 16)] = jnp.zeros((16,), jnp.float32)
            return 0

        def _orow(i, _):
            ones_v[pl.ds(i * 16, 16)] = jnp.ones((16,), jnp.float32)
            wvec_v[pl.ds(i * 16, 16)] = jnp.zeros((16,), jnp.float32)
            return 0

        lax.fori_loop(0, NPT, _zrow, 0)
        lax.fori_loop(0, SCH // 16, _orow, 0)
        pltpu.sync_copy(rbuf_v.at[pl.ds(0, NPT)],
                        a64_sh.at[pl.ds(sid * NPT, NPT)])
        pltpu.sync_copy(wvec_v.at[pl.ds(0, NPT1)],
                        aw_sh.at[pl.ds(sid * NPT1, NPT1)])
        pltpu.sync_copy(wvec_v.at[pl.ds(0, NPT1)],
                        ac_sh.at[pl.ds(sid * NPT1, NPT1)])
        plsc.subcore_barrier()

        for ch in range(EPW // SCH):
            base = wid * EPW + ch * SCH
            pltpu.sync_copy(col_hbm.at[pl.ds(base, SCH)], idx_v)
            pltpu.sync_copy(r_hbm.at[pl.ds(base, SCH)], rbuf_v)
            pltpu.sync_copy(w_hbm.at[pl.ds(base, SCH)], wvec_v)
            pltpu.sync_copy(rbuf_v, a64_sh.at[idx_v], add=True)
            pltpu.sync_copy(wvec_v, aw_sh.at[idx_v], add=True)
            pltpu.sync_copy(ones_v, ac_sh.at[idx_v], add=True)

        plsc.subcore_barrier()
        pltpu.sync_copy(
            a64_sh.at[pl.ds(sid * NPT, NPT)],
            o64_hbm.at[cid, pl.ds(sid * NPT, NPT)],
        )
        pltpu.sync_copy(
            aw_sh.at[pl.ds(sid * NPT1, NPT1)],
            ow_hbm.at[cid, pl.ds(sid * NPT1, NPT1)],
        )
        pltpu.sync_copy(
            ac_sh.at[pl.ds(sid * NPT1, NPT1)],
            oc_hbm.at[cid, pl.ds(sid * NPT1, NPT1)],
        )

    return _sc_scatter


def kernel(x, edge_index, edge_attr, u, node_batch, edge_batch, wts,
           mW1, mb1, mg1, mbe1, mW2, mb2,
           nW1, nb1, ng1, nbe1, nW2, nb2):
    node_out = nW2.shape[1]
    row = edge_index[0]
    col = edge_index[1]

    w0 = jnp.concatenate([mW1[:128], nW1[:128]], axis=1)
    m64 = mW2 @ nW1[128:384]
    vb = mb2 @ nW1[128:384]
    u_proj = u @ nW1[384:]
    w1e = mW1[128:]
    z16 = jnp.zeros((16, 64), jnp.float32)
    z8 = jnp.zeros((8, 128), jnp.float32)
    w2p = jnp.concatenate([
        jnp.concatenate([w1e, jnp.zeros((16, 64), jnp.float32)], axis=1),
        jnp.concatenate([jnp.zeros((16, 64), jnp.float32), w1e], axis=1),
        jnp.zeros((8, 128), jnp.float32),
    ], axis=0)
    wselp = jnp.zeros((40, 128), jnp.float32)
    wselp = wselp.at[32, :F].set(1.0).at[33, F:].set(1.0)
    half = (jnp.arange(128) // F)
    hmm = jnp.where(half[:, None] == half[None, :], 1.0 / F, 0.0)
    c8 = jnp.concatenate([
        jnp.tile(mb1, 2)[None, :], jnp.tile(mg1, 2)[None, :],
        jnp.tile(mbe1, 2)[None, :], jnp.zeros((5, 128), jnp.float32)], axis=0)
    z = jnp.zeros((1, F), jnp.float32)
    v8 = jnp.concatenate(
        [vb[None, :], nb1[None, :], ng1[None, :], nbe1[None, :]] + [z] * 4,
        axis=0)
    zb = jnp.zeros((1, node_out), jnp.float32)
    b2r = jnp.concatenate([nb2[None, :]] + [zb] * 7, axis=0)

    oh = (node_batch[:, None] == jnp.arange(16)[None, :]).astype(jnp.float32)

    col_p = col.reshape(2, EH).T.reshape(E)
    wts_p = wts.reshape(2, EH).T.reshape(E)

    ea_t = edge_attr.T
    w_t = wts.T
    eawt = jnp.concatenate([
        ea_t[:, :EH], ea_t[:, EH:], w_t[:, :EH], w_t[:, EH:],
        jnp.zeros((6, EH), jnp.float32)], axis=0)


    a, xn = _p0(x, w0)
    ag2 = _make_sc_gather()(a, row)
    r2 = _p2(ag2, eawt, w2p, wselp, hmm, c8)
    r_lin = r2.reshape(E, F)
    s64, sw, sc = _make_sc_scatter()(r_lin, col_p, wts_p)
    return _p4(s64, sw[:, :N].reshape(NC, N, 1), sc[:, :N].reshape(NC, N, 1),
               xn, oh, u_proj, m64, v8, nW2, b2r, node_out)


# speedup vs baseline: 1.2631x; 1.2631x over previous
import functools

import jax
import jax.numpy as jnp
from jax import lax
from jax.experimental import pallas as pl
from jax.experimental.pallas import tpu as pltpu
from jax.experimental.pallas import tpu_sc as plsc

N = 10000
E = 320000
EH = E // 2
F = 64
SW = 80
NC = 2
NS = 16
NW = NC * NS
PPW = EH // NW
EPW = E // NW
GCH = 1000
SCH = 200
NPT = N // NS
N2 = 10240
NPT1 = N2 // NS

BN = 1000
BP = 6400


def _p0_body(x_ref, w_ref, a_ref, xn_ref):
    r = jnp.dot(x_ref[...], w_ref[...], preferred_element_type=jnp.float32)
    a_ref[...] = r[:, :F]
    xn_ref[...] = r[:, F:]


def _p0(x, w0):
    return pl.pallas_call(
        _p0_body,
        grid=(N // BN,),
        in_specs=[
            pl.BlockSpec((BN, 128), lambda i: (i, 0)),
            pl.BlockSpec((128, 128), lambda i: (0, 0)),
        ],
        out_specs=[
            pl.BlockSpec((BN, F), lambda i: (i, 0)),
            pl.BlockSpec((BN, F), lambda i: (i, 0)),
        ],
        out_shape=[
            jax.ShapeDtypeStruct((N, F), jnp.float32),
            jax.ShapeDtypeStruct((N, F), jnp.float32),
        ],
    )(x, w0)


def _p2_body(ag_ref, et_ref, w2_ref, ws_ref, hm_ref, c_ref, oe_ref, oo_ref):
    et = et_ref[...]
    bp = jax.lax.dot_general(
        et, w2_ref[...], (((0,), (0,)), ((), ())),
        preferred_element_type=jnp.float32)
    wfull = jax.lax.dot_general(
        et, ws_ref[...], (((0,), (0,)), ((), ())),
        preferred_element_type=jnp.float32)
    h = ag_ref[...] + bp + c_ref[0:1, :]
    hm = hm_ref[...]
    mu = jnp.dot(h, hm, preferred_element_type=jnp.float32)
    hc = h - mu
    var = jnp.dot(hc * hc, hm, preferred_element_type=jnp.float32)
    hn = hc * lax.rsqrt(var + 1e-5) * c_ref[1:2, :] + c_ref[2:3, :]
    rw = jnp.maximum(hn, 0.0) * wfull
    lane = lax.broadcasted_iota(jnp.int32, (1, 128), 1) % F
    wr = jnp.where(lane == 0, wfull, 0.0) + jnp.where(lane == 1, 1.0, 0.0)
    oe_ref[...] = jnp.concatenate([rw[:, :F], wr[:, :F]], axis=1)
    oo_ref[...] = jnp.concatenate([rw[:, F:], wr[:, F:]], axis=1)


def _p2(ag2, eawt, w2, wsel, hm, c8):
    return pl.pallas_call(
        _p2_body,
        grid=(EH // BP,),
        in_specs=[
            pl.BlockSpec((BP, 128), lambda i: (i, 0)),
            pl.BlockSpec((40, BP), lambda i: (0, i)),
            pl.BlockSpec((40, 128), lambda i: (0, 0)),
            pl.BlockSpec((40, 128), lambda i: (0, 0)),
            pl.BlockSpec((128, 128), lambda i: (0, 0)),
            pl.BlockSpec((8, 128), lambda i: (0, 0)),
        ],
        out_specs=[
            pl.BlockSpec((BP, 128), lambda i: (i, 0)),
            pl.BlockSpec((BP, 128), lambda i: (i, 0)),
        ],
        out_shape=[
            jax.ShapeDtypeStruct((EH, 128), jnp.float32),
            jax.ShapeDtypeStruct((EH, 128), jnp.float32),
        ],
    )(ag2, eawt, w2, wsel, hm, c8)


def _p4_body(s2_ref, xn_ref, oh_ref, up_ref, m_ref, v_ref,
             w2_ref, b2_ref, out_ref):
    s = s2_ref[0] + s2_ref[1]
    sm = s[:, :F]
    wsum = s[:, F:F + 1]
    cnt = s[:, F + 1:F + 2]
    recv = (
        jnp.dot(sm, m_ref[...], preferred_element_type=jnp.float32)
        + wsum * v_ref[0:1, :]
    ) / jnp.maximum(cnt, 1.0)
    h = (
        xn_ref[...] + recv
        + jnp.dot(oh_ref[...], up_ref[...], preferred_element_type=jnp.float32)
        + v_ref[1:2, :]
    )
    mu = jnp.mean(h, axis=-1, keepdims=True)
    var = jnp.mean((h - mu) ** 2, axis=-1, keepdims=True)
    hn = (h - mu) * lax.rsqrt(var + 1e-5) * v_ref[2:3, :] + v_ref[3:4, :]
    hr = jnp.maximum(hn, 0.0)
    out_ref[...] = (
        jnp.dot(hr, w2_ref[...], preferred_element_type=jnp.float32)
        + b2_ref[0:1, :]
    )


def _p4(s2, xn, oh, up, m64, v8, nw2, b2r, node_out):
    return pl.pallas_call(
        _p4_body,
        grid=(N // BN,),
        in_specs=[
            pl.BlockSpec((NC, BN, 128), lambda i: (0, i, 0)),
            pl.BlockSpec((BN, F), lambda i: (i, 0)),
            pl.BlockSpec((BN, 16), lambda i: (i, 0)),
            pl.BlockSpec((16, F), lambda i: (0, 0)),
            pl.BlockSpec((F, F), lambda i: (0, 0)),
            pl.BlockSpec((8, F), lambda i: (0, 0)),
            pl.BlockSpec((F, node_out), lambda i: (0, 0)),
            pl.BlockSpec((8, node_out), lambda i: (0, 0)),
        ],
        out_specs=pl.BlockSpec((BN, node_out), lambda i: (i, 0)),
        out_shape=jax.ShapeDtypeStruct((N, node_out), jnp.float32),
    )(s2, xn, oh, up, m64, v8, nw2, b2r)


@functools.lru_cache(maxsize=None)
def _make_sc_gather():
    mesh = plsc.VectorSubcoreMesh(core_axis_name="c", subcore_axis_name="s")

    @functools.partial(
        pl.kernel,
        mesh=mesh,
        out_type=jax.ShapeDtypeStruct((EH, 128), jnp.float32),
        scratch_types=[
            pltpu.VMEM((GCH,), jnp.int32),
            pltpu.VMEM((GCH,), jnp.int32),
            pltpu.VMEM((GCH, F), jnp.float32),
            pltpu.VMEM((GCH, F), jnp.float32),
            pltpu.SemaphoreType.DMA,
            pltpu.SemaphoreType.DMA,
            pltpu.SemaphoreType.DMA,
            pltpu.SemaphoreType.DMA,
        ],
        compiler_params=pltpu.CompilerParams(use_tc_tiling_on_sc=False),
    )
    def _sc_gather(a_hbm, row_hbm, out_hbm, idx0_v, idx1_v, rows0_v, rows1_v,
                   semg0, semg1, sems0, sems1):
        cid = lax.axis_index("c")
        sid = lax.axis_index("s")
        wid = cid * NS + sid
        idx = (idx0_v, idx1_v)
        rows = (rows0_v, rows1_v)
        semg = (semg0, semg1)
        sems = (sems0, sems1)
        steps = []
        for ch in range(PPW // GCH):
            for half in range(2):
                steps.append((wid * PPW + ch * GCH, half))
        nst = len(steps)
        g_h = [None, None]
        s_h = [None, None]
        for k in range(nst + 1):
            if k < nst:
                pbase, half = steps[k]
                b = k % 2
                if s_h[b] is not None:
                    s_h[b].wait()
                pltpu.sync_copy(row_hbm.at[pl.ds(half * EH + pbase, GCH)],
                                idx[b])
                g_h[b] = pltpu.async_copy(a_hbm.at[idx[b]], rows[b], semg[b])
            if k >= 1:
                ppbase, phalf = steps[k - 1]
                pb = (k - 1) % 2
                g_h[pb].wait()
                s_h[pb] = pltpu.async_copy(
                    rows[pb],
                    out_hbm.at[pl.ds(ppbase, GCH), pl.ds(phalf * F, F)],
                    sems[pb])
        s_h[(nst - 1) % 2].wait()

    return _sc_gather


@functools.lru_cache(maxsize=None)
def _make_sc_scatter():
    mesh = plsc.VectorSubcoreMesh(core_axis_name="c", subcore_axis_name="s")

    @functools.partial(
        pl.kernel,
        mesh=mesh,
        out_type=jax.ShapeDtypeStruct((NC, N, 128), jnp.float32),
        scratch_types=[
            pltpu.VMEM((SCH,), jnp.int32),
            pltpu.VMEM((SCH, 128), jnp.float32),
            pltpu.VMEM_SHARED((N, 128), jnp.float32),
            pltpu.SemaphoreType.DMA,
        ],
        compiler_params=pltpu.CompilerParams(use_tc_tiling_on_sc=False),
    )
    def _sc_scatter(re_hbm, ro_hbm, col_hbm, out_hbm,
                    idx_v, rbuf_v, acc_sh, sem):
        cid = lax.axis_index("c")
        sid = lax.axis_index("s")
        wid = cid * NS + sid

        def _zrow(i, _):
            for j in range(128 // 16):
                rbuf_v[i, pl.ds(j * 16, 16)] = jnp.zeros((16,), jnp.float32)
            return 0

        lax.fori_loop(0, SCH, _zrow, 0)
        for part in range(NPT // SCH + 1):
            size = NPT - part * SCH if part == NPT // SCH else SCH
            if size > 0:
                pltpu.sync_copy(
                    rbuf_v.at[pl.ds(0, size)],
                    acc_sh.at[pl.ds(sid * NPT + part * SCH, size)])
        plsc.subcore_barrier()

        nch = PPW // SCH
        for half, r_hbm in ((0, re_hbm), (1, ro_hbm)):
            for ch in range(nch):
                pbase = wid * PPW + ch * SCH
                pltpu.sync_copy(col_hbm.at[pl.ds(half * EH + pbase, SCH)],
                                idx_v)
                pltpu.sync_copy(r_hbm.at[pl.ds(pbase, SCH)], rbuf_v)
                pltpu.sync_copy(rbuf_v, acc_sh.at[idx_v], add=True)

        plsc.subcore_barrier()
        pltpu.sync_copy(
            acc_sh.at[pl.ds(sid * NPT, NPT)],
            out_hbm.at[cid, pl.ds(sid * NPT, NPT)],
        )

    return _sc_scatter


def kernel(x, edge_index, edge_attr, u, node_batch, edge_batch, wts,
           mW1, mb1, mg1, mbe1, mW2, mb2,
           nW1, nb1, ng1, nbe1, nW2, nb2):
    node_out = nW2.shape[1]
    row = edge_index[0]
    col = edge_index[1]

    w0 = jnp.concatenate([mW1[:128], nW1[:128]], axis=1)
    m64 = mW2 @ nW1[128:384]
    vb = mb2 @ nW1[128:384]
    u_proj = u @ nW1[384:]
    w1e = mW1[128:]
    z16 = jnp.zeros((16, 64), jnp.float32)
    z8 = jnp.zeros((8, 128), jnp.float32)
    w2p = jnp.concatenate([
        jnp.concatenate([w1e, jnp.zeros((16, 64), jnp.float32)], axis=1),
        jnp.concatenate([jnp.zeros((16, 64), jnp.float32), w1e], axis=1),
        jnp.zeros((8, 128), jnp.float32),
    ], axis=0)
    wselp = jnp.zeros((40, 128), jnp.float32)
    wselp = wselp.at[32, :F].set(1.0).at[33, F:].set(1.0)
    half = (jnp.arange(128) // F)
    hmm = jnp.where(half[:, None] == half[None, :], 1.0 / F, 0.0)
    c8 = jnp.concatenate([
        jnp.tile(mb1, 2)[None, :], jnp.tile(mg1, 2)[None, :],
        jnp.tile(mbe1, 2)[None, :], jnp.zeros((5, 128), jnp.float32)], axis=0)
    z = jnp.zeros((1, F), jnp.float32)
    v8 = jnp.concatenate(
        [vb[None, :], nb1[None, :], ng1[None, :], nbe1[None, :]] + [z] * 4,
        axis=0)
    zb = jnp.zeros((1, node_out), jnp.float32)
    b2r = jnp.concatenate([nb2[None, :]] + [zb] * 7, axis=0)

    oh = (node_batch[:, None] == jnp.arange(16)[None, :]).astype(jnp.float32)


    ea_t = edge_attr.T
    w_t = wts.T
    eawt = jnp.concatenate([
        ea_t[:, :EH], ea_t[:, EH:], w_t[:, :EH], w_t[:, EH:],
        jnp.zeros((6, EH), jnp.float32)], axis=0)


    a, xn = _p0(x, w0)
    ag2 = _make_sc_gather()(a, row)
    r_e, r_o = _p2(ag2, eawt, w2p, wselp, hmm, c8)
    s2 = _make_sc_scatter()(r_e, r_o, col)
    return _p4(s2, xn, oh, u_proj, m64, v8, nW2, b2r, node_out)


# speedup vs baseline: 1.4386x; 1.1389x over previous
import functools

import jax
import jax.numpy as jnp
from jax import lax
from jax.experimental import pallas as pl
from jax.experimental.pallas import tpu as pltpu
from jax.experimental.pallas import tpu_sc as plsc

N = 10000
E = 320000
EH = E // 2
F = 64
SW = 80
NC = 2
NS = 16
NW = NC * NS
PPW = EH // NW
EPW = E // NW
GCH = 1000
SCH = 1000
NPT = N // NS
N2 = 10240
NPT1 = N2 // NS

BN = 1000
BP = 6400


def _p0_body(x_ref, w_ref, a_ref, xn_ref):
    r = jnp.dot(x_ref[...], w_ref[...], preferred_element_type=jnp.float32)
    a_ref[...] = r[:, :F]
    xn_ref[...] = r[:, F:]


def _p0(x, w0):
    return pl.pallas_call(
        _p0_body,
        grid=(N // BN,),
        in_specs=[
            pl.BlockSpec((BN, 128), lambda i: (i, 0)),
            pl.BlockSpec((128, 128), lambda i: (0, 0)),
        ],
        out_specs=[
            pl.BlockSpec((BN, F), lambda i: (i, 0)),
            pl.BlockSpec((BN, F), lambda i: (i, 0)),
        ],
        out_shape=[
            jax.ShapeDtypeStruct((N, F), jnp.float32),
            jax.ShapeDtypeStruct((N, F), jnp.float32),
        ],
    )(x, w0)


def _p2_body(ag_ref, et_ref, w2_ref, ws_ref, hm_ref, c_ref, oe_ref, oo_ref):
    et = et_ref[...]
    bp = jax.lax.dot_general(
        et, w2_ref[...], (((0,), (0,)), ((), ())),
        preferred_element_type=jnp.float32)
    wfull = jax.lax.dot_general(
        et, ws_ref[...], (((0,), (0,)), ((), ())),
        preferred_element_type=jnp.float32)
    h = ag_ref[...] + bp + c_ref[0:1, :]
    hm = hm_ref[...]
    mu = jnp.dot(h, hm, preferred_element_type=jnp.float32)
    hc = h - mu
    var = jnp.dot(hc * hc, hm, preferred_element_type=jnp.float32)
    hn = hc * lax.rsqrt(var + 1e-5) * c_ref[1:2, :] + c_ref[2:3, :]
    rw = jnp.maximum(hn, 0.0) * wfull
    lane = lax.broadcasted_iota(jnp.int32, (1, 128), 1) % F
    wr = jnp.where(lane == 0, wfull, 0.0) + jnp.where(lane == 1, 1.0, 0.0)
    oe_ref[...] = jnp.concatenate([rw[:, :F], wr[:, :F]], axis=1)
    oo_ref[...] = jnp.concatenate([rw[:, F:], wr[:, F:]], axis=1)


def _p2(ag2, eawt, w2, wsel, hm, c8):
    return pl.pallas_call(
        _p2_body,
        grid=(EH // BP,),
        in_specs=[
            pl.BlockSpec((BP, 128), lambda i: (i, 0)),
            pl.BlockSpec((40, BP), lambda i: (0, i)),
            pl.BlockSpec((40, 128), lambda i: (0, 0)),
            pl.BlockSpec((40, 128), lambda i: (0, 0)),
            pl.BlockSpec((128, 128), lambda i: (0, 0)),
            pl.BlockSpec((8, 128), lambda i: (0, 0)),
        ],
        out_specs=[
            pl.BlockSpec((BP, 128), lambda i: (i, 0)),
            pl.BlockSpec((BP, 128), lambda i: (i, 0)),
        ],
        out_shape=[
            jax.ShapeDtypeStruct((EH, 128), jnp.float32),
            jax.ShapeDtypeStruct((EH, 128), jnp.float32),
        ],
    )(ag2, eawt, w2, wsel, hm, c8)


def _p4_body(s64_ref, s16_ref, xn_ref, oh_ref, up_ref, m_ref, v_ref,
             w2_ref, b2_ref, out_ref):
    sm = s64_ref[0] + s64_ref[1]
    wc = s16_ref[0] + s16_ref[1]
    wsum = wc[:, 0:1]
    cnt = wc[:, 1:2]
    recv = (
        jnp.dot(sm, m_ref[...], preferred_element_type=jnp.float32)
        + wsum * v_ref[0:1, :]
    ) / jnp.maximum(cnt, 1.0)
    h = (
        xn_ref[...] + recv
        + jnp.dot(oh_ref[...], up_ref[...], preferred_element_type=jnp.float32)
        + v_ref[1:2, :]
    )
    mu = jnp.mean(h, axis=-1, keepdims=True)
    var = jnp.mean((h - mu) ** 2, axis=-1, keepdims=True)
    hn = (h - mu) * lax.rsqrt(var + 1e-5) * v_ref[2:3, :] + v_ref[3:4, :]
    hr = jnp.maximum(hn, 0.0)
    out_ref[...] = (
        jnp.dot(hr, w2_ref[...], preferred_element_type=jnp.float32)
        + b2_ref[0:1, :]
    )


def _p4(s64, s16, xn, oh, up, m64, v8, nw2, b2r, node_out):
    return pl.pallas_call(
        _p4_body,
        grid=(N // BN,),
        in_specs=[
            pl.BlockSpec((NC, BN, F), lambda i: (0, i, 0)),
            pl.BlockSpec((NC, BN, 16), lambda i: (0, i, 0)),
            pl.BlockSpec((BN, F), lambda i: (i, 0)),
            pl.BlockSpec((BN, 16), lambda i: (i, 0)),
            pl.BlockSpec((16, F), lambda i: (0, 0)),
            pl.BlockSpec((F, F), lambda i: (0, 0)),
            pl.BlockSpec((8, F), lambda i: (0, 0)),
            pl.BlockSpec((F, node_out), lambda i: (0, 0)),
            pl.BlockSpec((8, node_out), lambda i: (0, 0)),
        ],
        out_specs=pl.BlockSpec((BN, node_out), lambda i: (i, 0)),
        out_shape=jax.ShapeDtypeStruct((N, node_out), jnp.float32),
    )(s64, s16, xn, oh, up, m64, v8, nw2, b2r)


@functools.lru_cache(maxsize=None)
def _make_sc_gather():
    mesh = plsc.VectorSubcoreMesh(core_axis_name="c", subcore_axis_name="s")

    @functools.partial(
        pl.kernel,
        mesh=mesh,
        out_type=jax.ShapeDtypeStruct((EH, 128), jnp.float32),
        scratch_types=[
            pltpu.VMEM((GCH,), jnp.int32),
            pltpu.VMEM((GCH,), jnp.int32),
            pltpu.VMEM((GCH, F), jnp.float32),
            pltpu.VMEM((GCH, F), jnp.float32),
            pltpu.SemaphoreType.DMA,
            pltpu.SemaphoreType.DMA,
            pltpu.SemaphoreType.DMA,
            pltpu.SemaphoreType.DMA,
        ],
        compiler_params=pltpu.CompilerParams(use_tc_tiling_on_sc=False),
    )
    def _sc_gather(a_hbm, row_hbm, out_hbm, idx0_v, idx1_v, rows0_v, rows1_v,
                   semg0, semg1, sems0, sems1):
        cid = lax.axis_index("c")
        sid = lax.axis_index("s")
        wid = cid * NS + sid
        idx = (idx0_v, idx1_v)
        rows = (rows0_v, rows1_v)
        semg = (semg0, semg1)
        sems = (sems0, sems1)
        steps = []
        for ch in range(PPW // GCH):
            for half in range(2):
                steps.append((wid * PPW + ch * GCH, half))
        nst = len(steps)
        g_h = [None, None]
        s_h = [None, None]
        for k in range(nst + 1):
            if k < nst:
                pbase, half = steps[k]
                b = k % 2
                if s_h[b] is not None:
                    s_h[b].wait()
                pltpu.sync_copy(row_hbm.at[pl.ds(half * EH + pbase, GCH)],
                                idx[b])
                g_h[b] = pltpu.async_copy(a_hbm.at[idx[b]], rows[b], semg[b])
            if k >= 1:
                ppbase, phalf = steps[k - 1]
                pb = (k - 1) % 2
                g_h[pb].wait()
                s_h[pb] = pltpu.async_copy(
                    rows[pb],
                    out_hbm.at[pl.ds(ppbase, GCH), pl.ds(phalf * F, F)],
                    sems[pb])
        s_h[(nst - 1) % 2].wait()

    return _sc_gather


@functools.lru_cache(maxsize=None)
def _make_sc_scatter():
    mesh = plsc.VectorSubcoreMesh(core_axis_name="c", subcore_axis_name="s")

    @functools.partial(
        pl.kernel,
        mesh=mesh,
        out_type=[
            jax.ShapeDtypeStruct((NC, N, F), jnp.float32),
            jax.ShapeDtypeStruct((NC, N, 16), jnp.float32),
        ],
        scratch_types=[
            pltpu.VMEM((SCH,), jnp.int32),
            pltpu.VMEM((SCH, F), jnp.float32),
            pltpu.VMEM((SCH, 16), jnp.float32),
            pltpu.VMEM_SHARED((N, F), jnp.float32),
            pltpu.VMEM_SHARED((N, 16), jnp.float32),
            pltpu.SemaphoreType.DMA,
        ],
        compiler_params=pltpu.CompilerParams(use_tc_tiling_on_sc=False),
    )
    def _sc_scatter(re_hbm, ro_hbm, col_hbm, o64_hbm, o16_hbm,
                    idx_v, rbuf_v, wbuf_v, a64_sh, a16_sh, sem):
        cid = lax.axis_index("c")
        sid = lax.axis_index("s")
        wid = cid * NS + sid

        def _zrow(i, _):
            for j in range(F // 16):
                rbuf_v[i, pl.ds(j * 16, 16)] = jnp.zeros((16,), jnp.float32)
            wbuf_v[i, pl.ds(0, 16)] = jnp.zeros((16,), jnp.float32)
            return 0

        lax.fori_loop(0, NPT, _zrow, 0)
        pltpu.sync_copy(rbuf_v.at[pl.ds(0, NPT)],
                        a64_sh.at[pl.ds(sid * NPT, NPT)])
        pltpu.sync_copy(wbuf_v.at[pl.ds(0, NPT)],
                        a16_sh.at[pl.ds(sid * NPT, NPT)])
        plsc.subcore_barrier()

        nch = PPW // SCH
        for half, r_hbm in ((0, re_hbm), (1, ro_hbm)):
            for ch in range(nch):
                pbase = wid * PPW + ch * SCH
                pltpu.sync_copy(col_hbm.at[pl.ds(half * EH + pbase, SCH)],
                                idx_v)
                pltpu.sync_copy(
                    r_hbm.at[pl.ds(pbase, SCH), pl.ds(0, F)], rbuf_v)
                pltpu.sync_copy(
                    r_hbm.at[pl.ds(pbase, SCH), pl.ds(F, 16)], wbuf_v)
                pltpu.sync_copy(rbuf_v, a64_sh.at[idx_v], add=True)
                pltpu.sync_copy(wbuf_v, a16_sh.at[idx_v], add=True)

        plsc.subcore_barrier()
        pltpu.sync_copy(
            a64_sh.at[pl.ds(sid * NPT, NPT)],
            o64_hbm.at[cid, pl.ds(sid * NPT, NPT)],
        )
        pltpu.sync_copy(
            a16_sh.at[pl.ds(sid * NPT, NPT)],
            o16_hbm.at[cid, pl.ds(sid * NPT, NPT)],
        )

    return _sc_scatter


def kernel(x, edge_index, edge_attr, u, node_batch, edge_batch, wts,
           mW1, mb1, mg1, mbe1, mW2, mb2,
           nW1, nb1, ng1, nbe1, nW2, nb2):
    node_out = nW2.shape[1]
    row = edge_index[0]
    col = edge_index[1]

    w0 = jnp.concatenate([mW1[:128], nW1[:128]], axis=1)
    m64 = mW2 @ nW1[128:384]
    vb = mb2 @ nW1[128:384]
    u_proj = u @ nW1[384:]
    w1e = mW1[128:]
    z16 = jnp.zeros((16, 64), jnp.float32)
    z8 = jnp.zeros((8, 128), jnp.float32)
    w2p = jnp.concatenate([
        jnp.concatenate([w1e, jnp.zeros((16, 64), jnp.float32)], axis=1),
        jnp.concatenate([jnp.zeros((16, 64), jnp.float32), w1e], axis=1),
        jnp.zeros((8, 128), jnp.float32),
    ], axis=0)
    wselp = jnp.zeros((40, 128), jnp.float32)
    wselp = wselp.at[32, :F].set(1.0).at[33, F:].set(1.0)
    half = (jnp.arange(128) // F)
    hmm = jnp.where(half[:, None] == half[None, :], 1.0 / F, 0.0)
    c8 = jnp.concatenate([
        jnp.tile(mb1, 2)[None, :], jnp.tile(mg1, 2)[None, :],
        jnp.tile(mbe1, 2)[None, :], jnp.zeros((5, 128), jnp.float32)], axis=0)
    z = jnp.zeros((1, F), jnp.float32)
    v8 = jnp.concatenate(
        [vb[None, :], nb1[None, :], ng1[None, :], nbe1[None, :]] + [z] * 4,
        axis=0)
    zb = jnp.zeros((1, node_out), jnp.float32)
    b2r = jnp.concatenate([nb2[None, :]] + [zb] * 7, axis=0)

    oh = (node_batch[:, None] == jnp.arange(16)[None, :]).astype(jnp.float32)


    ea_t = edge_attr.T
    w_t = wts.T
    eawt = jnp.concatenate([
        ea_t[:, :EH], ea_t[:, EH:], w_t[:, :EH], w_t[:, EH:],
        jnp.zeros((6, EH), jnp.float32)], axis=0)


    a, xn = _p0(x, w0)
    ag2 = _make_sc_gather()(a, row)
    r_e, r_o = _p2(ag2, eawt, w2p, wselp, hmm, c8)
    s64, s16 = _make_sc_scatter()(r_e, r_o, col)
    return _p4(s64, s16, xn, oh, u_proj, m64, v8, nW2, b2r, node_out)


# speedup vs baseline: 1.4389x; 1.0003x over previous
import functools

import jax
import jax.numpy as jnp
from jax import lax
from jax.experimental import pallas as pl
from jax.experimental.pallas import tpu as pltpu
from jax.experimental.pallas import tpu_sc as plsc

N = 10000
E = 320000
EH = E // 2
F = 64
NC = 2
NS = 16
NW = NC * NS
PPW = EH // NW
EPW = E // NW
GCH = 1000
SCH = 1000
NPT = N // NS

BN = 1000
BP = 6400


def _p0_body(x_ref, w_ref, a_ref, xn_ref):
    r = jnp.dot(x_ref[...], w_ref[...], preferred_element_type=jnp.float32)
    a_ref[...] = r[:, :F]
    xn_ref[...] = r[:, F:]


def _p0(x, w0):
    return pl.pallas_call(
        _p0_body,
        grid=(N // BN,),
        in_specs=[
            pl.BlockSpec((BN, 128), lambda i: (i, 0)),
            pl.BlockSpec((128, 128), lambda i: (0, 0)),
        ],
        out_specs=[
            pl.BlockSpec((BN, F), lambda i: (i, 0)),
            pl.BlockSpec((BN, F), lambda i: (i, 0)),
        ],
        out_shape=[
            jax.ShapeDtypeStruct((N, F), jnp.float32),
            jax.ShapeDtypeStruct((N, F), jnp.float32),
        ],
    )(x, w0)


def _p2_body(ag_ref, et_ref, w2_ref, ws_ref, hm_ref, c_ref, oe_ref, oo_ref):
    et = et_ref[...]
    bp = jax.lax.dot_general(
        et, w2_ref[...], (((0,), (0,)), ((), ())),
        preferred_element_type=jnp.float32)
    wfull = jax.lax.dot_general(
        et, ws_ref[...], (((0,), (0,)), ((), ())),
        preferred_element_type=jnp.float32)
    h = ag_ref[...] + bp + c_ref[0:1, :]
    hm = hm_ref[...]
    mu = jnp.dot(h, hm, preferred_element_type=jnp.float32)
    hc = h - mu
    var = jnp.dot(hc * hc, hm, preferred_element_type=jnp.float32)
    hn = hc * lax.rsqrt(var + 1e-5) * c_ref[1:2, :] + c_ref[2:3, :]
    rw = jnp.maximum(hn, 0.0) * wfull
    lane = lax.broadcasted_iota(jnp.int32, (1, 128), 1) % F
    wr = jnp.where(lane == 0, wfull, 0.0) + jnp.where(lane == 1, 1.0, 0.0)
    oe_ref[...] = jnp.concatenate([rw[:, :F], wr[:, :F]], axis=1)
    oo_ref[...] = jnp.concatenate([rw[:, F:], wr[:, F:]], axis=1)


def _p2(ag2, eawt, w2, wsel, hm, c8):
    return pl.pallas_call(
        _p2_body,
        grid=(EH // BP,),
        in_specs=[
            pl.BlockSpec((BP, 128), lambda i: (i, 0)),
            pl.BlockSpec((40, BP), lambda i: (0, i)),
            pl.BlockSpec((40, 128), lambda i: (0, 0)),
            pl.BlockSpec((40, 128), lambda i: (0, 0)),
            pl.BlockSpec((128, 128), lambda i: (0, 0)),
            pl.BlockSpec((8, 128), lambda i: (0, 0)),
        ],
        out_specs=[
            pl.BlockSpec((BP, 128), lambda i: (i, 0)),
            pl.BlockSpec((BP, 128), lambda i: (i, 0)),
        ],
        out_shape=[
            jax.ShapeDtypeStruct((EH, 128), jnp.float32),
            jax.ShapeDtypeStruct((EH, 128), jnp.float32),
        ],
    )(ag2, eawt, w2, wsel, hm, c8)


def _p4_body(s64_ref, s16_ref, xn_ref, oh_ref, up_ref, m_ref, v_ref,
             w2_ref, b2_ref, out_ref):
    sm = s64_ref[0] + s64_ref[1]
    wc = s16_ref[0] + s16_ref[1]
    wsum = wc[:, 0:1]
    cnt = wc[:, 1:2]
    recv = (
        jnp.dot(sm, m_ref[...], preferred_element_type=jnp.float32)
        + wsum * v_ref[0:1, :]
    ) / jnp.maximum(cnt, 1.0)
    h = (
        xn_ref[...] + recv
        + jnp.dot(oh_ref[...], up_ref[...], preferred_element_type=jnp.float32)
        + v_ref[1:2, :]
    )
    mu = jnp.mean(h, axis=-1, keepdims=True)
    var = jnp.mean((h - mu) ** 2, axis=-1, keepdims=True)
    hn = (h - mu) * lax.rsqrt(var + 1e-5) * v_ref[2:3, :] + v_ref[3:4, :]
    hr = jnp.maximum(hn, 0.0)
    out_ref[...] = (
        jnp.dot(hr, w2_ref[...], preferred_element_type=jnp.float32)
        + b2_ref[0:1, :]
    )


def _p4(s64, s16, xn, oh, up, m64, v8, nw2, b2r, node_out):
    return pl.pallas_call(
        _p4_body,
        grid=(N // BN,),
        in_specs=[
            pl.BlockSpec((NC, BN, F), lambda i: (0, i, 0)),
            pl.BlockSpec((NC, BN, 16), lambda i: (0, i, 0)),
            pl.BlockSpec((BN, F), lambda i: (i, 0)),
            pl.BlockSpec((BN, 16), lambda i: (i, 0)),
            pl.BlockSpec((16, F), lambda i: (0, 0)),
            pl.BlockSpec((F, F), lambda i: (0, 0)),
            pl.BlockSpec((8, F), lambda i: (0, 0)),
            pl.BlockSpec((F, node_out), lambda i: (0, 0)),
            pl.BlockSpec((8, node_out), lambda i: (0, 0)),
        ],
        out_specs=pl.BlockSpec((BN, node_out), lambda i: (i, 0)),
        out_shape=jax.ShapeDtypeStruct((N, node_out), jnp.float32),
    )(s64, s16, xn, oh, up, m64, v8, nw2, b2r)


@functools.lru_cache(maxsize=None)
def _make_sc_gather():
    mesh = plsc.VectorSubcoreMesh(core_axis_name="c", subcore_axis_name="s")

    @functools.partial(
        pl.kernel,
        mesh=mesh,
        out_type=jax.ShapeDtypeStruct((EH, 128), jnp.float32),
        scratch_types=[
            pltpu.VMEM((GCH,), jnp.int32),
            pltpu.VMEM((GCH,), jnp.int32),
            pltpu.VMEM((GCH, F), jnp.float32),
            pltpu.VMEM((GCH, F), jnp.float32),
            pltpu.SemaphoreType.DMA,
            pltpu.SemaphoreType.DMA,
            pltpu.SemaphoreType.DMA,
            pltpu.SemaphoreType.DMA,
        ],
        compiler_params=pltpu.CompilerParams(use_tc_tiling_on_sc=False),
    )
    def _sc_gather(a_hbm, row_hbm, out_hbm, idx0_v, idx1_v, rows0_v, rows1_v,
                   semg0, semg1, sems0, sems1):
        cid = lax.axis_index("c")
        sid = lax.axis_index("s")
        wid = cid * NS + sid
        idx = (idx0_v, idx1_v)
        rows = (rows0_v, rows1_v)
        semg = (semg0, semg1)
        sems = (sems0, sems1)
        steps = []
        for ch in range(PPW // GCH):
            for half in range(2):
                steps.append((wid * PPW + ch * GCH, half))
        nst = len(steps)
        g_h = [None, None]
        s_h = [None, None]
        for k in range(nst + 1):
            if k < nst:
                pbase, half = steps[k]
                b = k % 2
                if s_h[b] is not None:
                    s_h[b].wait()
                pltpu.sync_copy(row_hbm.at[pl.ds(half * EH + pbase, GCH)],
                                idx[b])
                g_h[b] = pltpu.async_copy(a_hbm.at[idx[b]], rows[b], semg[b])
            if k >= 1:
                ppbase, phalf = steps[k - 1]
                pb = (k - 1) % 2
                g_h[pb].wait()
                s_h[pb] = pltpu.async_copy(
                    rows[pb],
                    out_hbm.at[pl.ds(ppbase, GCH), pl.ds(phalf * F, F)],
                    sems[pb])
        s_h[(nst - 1) % 2].wait()

    return _sc_gather


@functools.lru_cache(maxsize=None)
def _make_sc_scatter():
    mesh = plsc.VectorSubcoreMesh(core_axis_name="c", subcore_axis_name="s")

    @functools.partial(
        pl.kernel,
        mesh=mesh,
        out_type=[
            jax.ShapeDtypeStruct((NC, N, F), jnp.float32),
            jax.ShapeDtypeStruct((NC, N, 16), jnp.float32),
        ],
        scratch_types=[
            pltpu.VMEM((SCH,), jnp.int32),
            pltpu.VMEM((SCH, F), jnp.float32),
            pltpu.VMEM((SCH, 16), jnp.float32),
            pltpu.VMEM_SHARED((N, F), jnp.float32),
            pltpu.VMEM_SHARED((N, 16), jnp.float32),
            pltpu.SemaphoreType.DMA,
        ],
        compiler_params=pltpu.CompilerParams(use_tc_tiling_on_sc=False),
    )
    def _sc_scatter(re_hbm, ro_hbm, col_hbm, o64_hbm, o16_hbm,
                    idx_v, rbuf_v, wbuf_v, a64_sh, a16_sh, sem):
        cid = lax.axis_index("c")
        sid = lax.axis_index("s")
        wid = cid * NS + sid

        def _zrow(i, _):
            for j in range(F // 16):
                rbuf_v[i, pl.ds(j * 16, 16)] = jnp.zeros((16,), jnp.float32)
            wbuf_v[i, pl.ds(0, 16)] = jnp.zeros((16,), jnp.float32)
            return 0

        lax.fori_loop(0, NPT, _zrow, 0)
        pltpu.sync_copy(rbuf_v.at[pl.ds(0, NPT)],
                        a64_sh.at[pl.ds(sid * NPT, NPT)])
        pltpu.sync_copy(wbuf_v.at[pl.ds(0, NPT)],
                        a16_sh.at[pl.ds(sid * NPT, NPT)])
        plsc.subcore_barrier()

        nch = PPW // SCH
        for half, r_hbm in ((0, re_hbm), (1, ro_hbm)):
            for ch in range(nch):
                pbase = wid * PPW + ch * SCH
                pltpu.sync_copy(col_hbm.at[pl.ds(half * EH + pbase, SCH)],
                                idx_v)
                pltpu.sync_copy(
                    r_hbm.at[pl.ds(pbase, SCH), pl.ds(0, F)], rbuf_v)
                pltpu.sync_copy(
                    r_hbm.at[pl.ds(pbase, SCH), pl.ds(F, 16)], wbuf_v)
                pltpu.sync_copy(rbuf_v, a64_sh.at[idx_v], add=True)
                pltpu.sync_copy(wbuf_v, a16_sh.at[idx_v], add=True)

        plsc.subcore_barrier()
        pltpu.sync_copy(
            a64_sh.at[pl.ds(sid * NPT, NPT)],
            o64_hbm.at[cid, pl.ds(sid * NPT, NPT)],
        )
        pltpu.sync_copy(
            a16_sh.at[pl.ds(sid * NPT, NPT)],
            o16_hbm.at[cid, pl.ds(sid * NPT, NPT)],
        )

    return _sc_scatter


def kernel(x, edge_index, edge_attr, u, node_batch, edge_batch, wts,
           mW1, mb1, mg1, mbe1, mW2, mb2,
           nW1, nb1, ng1, nbe1, nW2, nb2):
    node_out = nW2.shape[1]
    row = edge_index[0]
    col = edge_index[1]

    w0 = jnp.concatenate([mW1[:128], nW1[:128]], axis=1)
    m64 = mW2 @ nW1[128:384]
    vb = mb2 @ nW1[128:384]
    u_proj = u @ nW1[384:]
    w1e = mW1[128:]
    z16 = jnp.zeros((16, 64), jnp.float32)
    z8 = jnp.zeros((8, 128), jnp.float32)
    w2p = jnp.concatenate([
        jnp.concatenate([w1e, jnp.zeros((16, 64), jnp.float32)], axis=1),
        jnp.concatenate([jnp.zeros((16, 64), jnp.float32), w1e], axis=1),
        jnp.zeros((8, 128), jnp.float32),
    ], axis=0)
    wselp = jnp.zeros((40, 128), jnp.float32)
    wselp = wselp.at[32, :F].set(1.0).at[33, F:].set(1.0)
    half = (jnp.arange(128) // F)
    hmm = jnp.where(half[:, None] == half[None, :], 1.0 / F, 0.0)
    c8 = jnp.concatenate([
        jnp.tile(mb1, 2)[None, :], jnp.tile(mg1, 2)[None, :],
        jnp.tile(mbe1, 2)[None, :], jnp.zeros((5, 128), jnp.float32)], axis=0)
    z = jnp.zeros((1, F), jnp.float32)
    v8 = jnp.concatenate(
        [vb[None, :], nb1[None, :], ng1[None, :], nbe1[None, :]] + [z] * 4,
        axis=0)
    zb = jnp.zeros((1, node_out), jnp.float32)
    b2r = jnp.concatenate([nb2[None, :]] + [zb] * 7, axis=0)

    oh = (node_batch[:, None] == jnp.arange(16)[None, :]).astype(jnp.float32)


    ea_t = edge_attr.T
    w_t = wts.T
    eawt = jnp.concatenate([
        ea_t[:, :EH], ea_t[:, EH:], w_t[:, :EH], w_t[:, EH:],
        jnp.zeros((6, EH), jnp.float32)], axis=0)


    a, xn = _p0(x, w0)
    ag2 = _make_sc_gather()(a, row)
    r_e, r_o = _p2(ag2, eawt, w2p, wselp, hmm, c8)
    s64, s16 = _make_sc_scatter()(r_e, r_o, col)
    return _p4(s64, s16, xn, oh, u_proj, m64, v8, nW2, b2r, node_out)
